# Initial kernel scaffold; baseline (speedup 1.0000x reference)
import jax
import jax.numpy as jnp
from jax.experimental import pallas as pl


def kernel(x, W_fc1, b_fc1, g_bn1, b_bn1, W_g, b_g, g_bng, b_bng, W_fc2, b_fc2, g_bn2, b_bn2):
    raise NotImplementedError("write your pallas kernel here")


# speedup vs baseline: 15.1587x; 15.1587x over previous
import numpy as np
import jax
import jax.numpy as jnp
from jax import lax
from jax.experimental import pallas as pl
from jax.experimental.pallas import tpu as pltpu

_B, _C, _H, _W = 8, 96, 32, 32
_N = _H * _W
_K = 9
_BN_ROWS = _B * _N
_EPS = 1e-5


def _sincos_1d_np(embed_dim, pos):
    omega = np.arange(embed_dim // 2, dtype=np.float64) / (embed_dim / 2.0)
    omega = 1.0 / (10000.0 ** omega)
    out = pos.reshape(-1)[:, None] * omega[None, :]
    return np.concatenate([np.sin(out), np.cos(out)], axis=1)


def _rel_pos_const(embed_dim, gh, gw):
    gx = np.arange(gw, dtype=np.float64)
    gy = np.arange(gh, dtype=np.float64)
    gX, gY = np.meshgrid(gx, gy)
    emb_w = _sincos_1d_np(embed_dim // 2, gX)
    emb_h = _sincos_1d_np(embed_dim // 2, gY)
    pos = np.concatenate([emb_h, emb_w], axis=1)
    rel = 2.0 * (pos @ pos.T) / float(embed_dim)
    return (-rel).astype(np.float32)


_REL_POS_NP = _rel_pos_const(_C, _H, _W)


def _bn_cols(v, gamma, beta):
    mu = jnp.mean(v, axis=0, keepdims=True)
    var = jnp.mean((v - mu) ** 2, axis=0, keepdims=True)
    return (v - mu) / jnp.sqrt(var + _EPS) * gamma + beta


def _dot_bf16(a, b, dims):
    return lax.dot_general(a.astype(jnp.bfloat16), b.astype(jnp.bfloat16),
                           dims, preferred_element_type=jnp.float32)


def _stage_a(xt_ref, w1_ref, b1_ref, g1_ref, bb1_ref, y_ref, yn_ref):
    xt = xt_ref[...]
    y_pre = _dot_bf16(xt, w1_ref[...], (((1,), (1,)), ((), ())))
    y_pre = y_pre + b1_ref[...]
    y = _bn_cols(y_pre, g1_ref[...], bb1_ref[...])
    nrm = jnp.sqrt(jnp.sum(y * y, axis=1, keepdims=True))
    yn = y / jnp.maximum(nrm, 1e-12)
    y_ref[...] = y
    yn_ref[...] = yn


def _stage_b(yn_ref, y_ref, rp_ref, out_ref):
    yn = yn_ref[...]
    y = y_ref[...]
    yn2 = yn * yn
    sq_col = jnp.sum(yn2, axis=1, keepdims=True)
    ones_row = jnp.ones((1, _C), jnp.float32)
    s_hi = yn2.astype(jnp.bfloat16)
    r1 = yn2 - s_hi.astype(jnp.float32)
    s_mid = r1.astype(jnp.bfloat16)
    s_lo = r1 - s_mid.astype(jnp.float32)
    dims_t = (((1,), (1,)), ((), ()))
    sq_row = (_dot_bf16(ones_row, s_hi, dims_t)
              + _dot_bf16(ones_row, s_mid, dims_t)
              + _dot_bf16(ones_row, s_lo, dims_t))
    g = _dot_bf16(yn, yn, dims_t)
    d = (sq_col - 2.0 * g) + (sq_row + rp_ref[...])
    y_hi = y.astype(jnp.bfloat16)
    y_lo = (y - y_hi.astype(jnp.float32)).astype(jnp.bfloat16)
    ycat = jnp.concatenate([y_hi, y_lo], axis=1)
    iota = lax.broadcasted_iota(jnp.int32, (_N, _N), 1)
    acc = jnp.full((_N, _C), -jnp.inf, jnp.float32)
    for _ in range(_K):
        m = jnp.min(d, axis=1, keepdims=True)
        idx = jnp.min(jnp.where(d == m, iota, _N), axis=1, keepdims=True)
        hit = iota == idx
        onehot = hit.astype(jnp.bfloat16)
        gk = lax.dot_general(onehot, ycat, (((1,), (0,)), ((), ())),
                             preferred_element_type=jnp.float32)
        acc = jnp.maximum(acc, gk[:, :_C] + gk[:, _C:])
        d = jnp.where(hit, jnp.inf, d)
    out_ref[...] = acc - y


def _stage_c(y_ref, rm_ref, xt_ref, wg_ref, bg_ref, gg_ref, bbg_ref,
             w2_ref, b2_ref, g2_ref, bb2_ref, out_ref):
    h = jnp.concatenate([y_ref[...], rm_ref[...]], axis=1)
    hg = _dot_bf16(h, wg_ref[...], (((1,), (0,)), ((), ()))) + bg_ref[...]
    hg = _bn_cols(hg, gg_ref[...], bbg_ref[...])
    hg = 0.5 * hg * (1.0 + lax.erf(hg * np.float32(1.0 / np.sqrt(2.0))))
    out = _dot_bf16(hg, w2_ref[...], (((1,), (1,)), ((), ()))) + b2_ref[...]
    out = _bn_cols(out, g2_ref[...], bb2_ref[...])
    out_ref[...] = out + xt_ref[...]


def kernel(x, W_fc1, b_fc1, g_bn1, b_bn1, W_g, b_g, g_bng, b_bng,
           W_fc2, b_fc2, g_bn2, b_bn2):
    xt = x.reshape(_B, _C, _N).transpose(0, 2, 1).reshape(_BN_ROWS, _C)
    rel_pos = jnp.asarray(_REL_POS_NP)

    y, yn = pl.pallas_call(
        _stage_a,
        out_shape=[jax.ShapeDtypeStruct((_BN_ROWS, _C), jnp.float32)] * 2,
    )(xt, W_fc1, b_fc1.reshape(1, _C), g_bn1.reshape(1, _C),
      b_bn1.reshape(1, _C))

    rel_max = pl.pallas_call(
        _stage_b,
        grid=(_B,),
        in_specs=[
            pl.BlockSpec((_N, _C), lambda i: (i, 0)),
            pl.BlockSpec((_N, _C), lambda i: (i, 0)),
            pl.BlockSpec((_N, _N), lambda i: (0, 0)),
        ],
        out_specs=pl.BlockSpec((_N, _C), lambda i: (i, 0)),
        out_shape=jax.ShapeDtypeStruct((_BN_ROWS, _C), jnp.float32),
        compiler_params=pltpu.CompilerParams(
            dimension_semantics=("arbitrary",)),
    )(yn, y, rel_pos)

    wg_bd = jax.scipy.linalg.block_diag(*[W_g[g].T for g in range(4)])

    out = pl.pallas_call(
        _stage_c,
        out_shape=jax.ShapeDtypeStruct((_BN_ROWS, _C), jnp.float32),
    )(y, rel_max, xt, wg_bd, b_g.reshape(1, 2 * _C), g_bng.reshape(1, 2 * _C),
      b_bng.reshape(1, 2 * _C), W_fc2, b_fc2.reshape(1, _C),
      g_bn2.reshape(1, _C), b_bn2.reshape(1, _C))

    return out.reshape(_B, _N, _C).transpose(0, 2, 1).reshape(_B, _C, _H, _W)
